# baseline (speedup 1.0000x reference)
import functools

import jax
import jax.numpy as jnp
from jax.experimental import pallas as pl
from jax.experimental.pallas import tpu as pltpu

_CW = 1280


def _phase_a_body(g_ref, x_ref, w1_ref, w2_ref, wd_ref, b_ref,
                  u_ref, pout_ref, usc_ref, *, bm, n):
    j = pl.program_id(0)
    g16 = g_ref[...].astype(jnp.bfloat16)
    t1 = jnp.dot(g16, x_ref[...].astype(jnp.bfloat16),
                 preferred_element_type=jnp.float32)
    u_j = jnp.dot(t1, w2_ref[...], preferred_element_type=jnp.float32)
    u16 = u_j.astype(jnp.bfloat16)

    @pl.when(j == 0)
    def _init():
        usc_ref[...] = jnp.zeros_like(usc_ref)

    usc_ref[pl.ds(j * bm, bm), :] = u16
    u_ref[...] = u16

    x_j = x_ref[pl.ds(j * bm, bm), :]
    pout_ref[...] = (jnp.dot(t1, w1_ref[...], preferred_element_type=jnp.float32)
                     + jnp.dot(x_j, wd_ref[...], preferred_element_type=jnp.float32)
                     + b_ref[...]
                     + 2.0 * jnp.dot(g16, usc_ref[...],
                                     preferred_element_type=jnp.float32))


def _phase_b_body(g_ref, u_ref, pout_ref, o_ref, *, bm, n, nc):
    j = pl.program_id(0)
    c = pl.program_id(1)
    c0 = ((j + 1) * bm) // _CW

    @pl.when(c == 0)
    def _init():
        o_ref[...] = pout_ref[...]

    @pl.when(c >= c0)
    def _acc():
        col = c * _CW + jax.lax.broadcasted_iota(jnp.int32, (bm, _CW), 1)
        keep = (col >= (j + 1) * bm) & (col < n)
        g16 = jnp.where(keep, g_ref[...], 0.0).astype(jnp.bfloat16)
        uc = u_ref[pl.ds(c * _CW, _CW), :]
        o_ref[...] += 2.0 * jnp.dot(g16, uc, preferred_element_type=jnp.float32)

    @pl.when(c == nc - 1)
    def _epilogue():
        pre = o_ref[...]
        m = jnp.max(pre, axis=1, keepdims=True)
        lse = jnp.log(jnp.sum(jnp.exp(pre - m), axis=1, keepdims=True)) + m
        o_ref[...] = pre - lse


def _pick_bm(n):
    for bm in (400, 200, 100, 80, 40, 16, 8):
        if n % bm == 0:
            return bm
    return n


@functools.partial(jax.jit, static_argnames=())
def kernel(x, gso, W, b):
    n, f = x.shape
    bm = _pick_bm(n)
    nj = n // bm
    nc = -(-n // _CW)
    npad = nc * _CW

    w0, w1, w2 = W[0], W[1], W[2]
    wd = (w0 - w2).astype(jnp.float32)
    b2 = b.reshape(1, f).astype(jnp.float32)

    row_blk = pl.BlockSpec((bm, n), lambda j: (j, 0))
    skinny_blk = pl.BlockSpec((bm, f), lambda j: (j, 0))
    full_w = pl.BlockSpec((f, f), lambda j: (0, 0))

    u, pout = pl.pallas_call(
        functools.partial(_phase_a_body, bm=bm, n=n),
        grid=(nj,),
        in_specs=[row_blk,
                  pl.BlockSpec((n, f), lambda j: (0, 0)),
                  full_w, full_w, full_w,
                  pl.BlockSpec((1, f), lambda j: (0, 0))],
        out_specs=[skinny_blk, skinny_blk],
        out_shape=[jax.ShapeDtypeStruct((n, f), jnp.bfloat16),
                   jax.ShapeDtypeStruct((n, f), jnp.float32)],
        scratch_shapes=[pltpu.VMEM((n, f), jnp.bfloat16)],
    )(gso, x, w1, w2, wd, b2)

    u_pad = jnp.pad(u, ((0, npad - n), (0, 0)))

    out = pl.pallas_call(
        functools.partial(_phase_b_body, bm=bm, n=n, nc=nc),
        grid=(nj, nc),
        in_specs=[pl.BlockSpec((bm, _CW),
                               lambda j, c: (j, jnp.maximum(c, ((j + 1) * bm) // _CW))),
                  pl.BlockSpec((npad, f), lambda j, c: (0, 0)),
                  pl.BlockSpec((bm, f), lambda j, c: (j, 0))],
        out_specs=pl.BlockSpec((bm, f), lambda j, c: (j, 0)),
        out_shape=jax.ShapeDtypeStruct((n, f), jnp.float32),
    )(gso, u_pad, pout)
    return out


# speedup vs baseline: 1.3137x; 1.3137x over previous
import functools

import jax
import jax.numpy as jnp
from jax.experimental import pallas as pl
from jax.experimental.pallas import tpu as pltpu

_CW = 1280


def _phase_a_body(g_ref, x_ref, w1_ref, w2_ref, wd_ref, b_ref,
                  u_ref, pout_ref, xu_ref, *, bm, n, f):
    j = pl.program_id(0)

    @pl.when(j == 0)
    def _init():
        xu_ref[:, :f] = x_ref[...].astype(jnp.bfloat16)
        xu_ref[:, f:] = jnp.zeros((n, f), jnp.bfloat16)

    g16 = g_ref[...].astype(jnp.bfloat16)
    both = jnp.dot(g16, xu_ref[...], preferred_element_type=jnp.float32)
    t1 = both[:, :f]
    partial = both[:, f:]

    u_j = jnp.dot(t1, w2_ref[...], preferred_element_type=jnp.float32)
    u16 = u_j.astype(jnp.bfloat16)
    xu_ref[pl.ds(j * bm, bm), f:] = u16
    u_ref[...] = u16

    x_j = xu_ref[pl.ds(j * bm, bm), :f]
    pout_ref[...] = (jnp.dot(t1, w1_ref[...], preferred_element_type=jnp.float32)
                     + jnp.dot(x_j, wd_ref[...], preferred_element_type=jnp.float32)
                     + b_ref[...]
                     + 2.0 * partial)


def _phase_b_body(g_ref, u_ref, pout_ref, o_ref, *, bm, n, nc):
    j = pl.program_id(0)
    c = pl.program_id(1)
    c0 = (j * bm) // _CW

    @pl.when(c == 0)
    def _init():
        o_ref[...] = pout_ref[...]

    @pl.when(c >= c0)
    def _acc():
        col = c * _CW + jax.lax.broadcasted_iota(jnp.int32, (bm, _CW), 1)
        keep = (col >= j * bm) & (col < n)
        g16 = jnp.where(keep, g_ref[...], 0.0).astype(jnp.bfloat16)
        uc = u_ref[pl.ds(c * _CW, _CW), :]
        o_ref[...] += 2.0 * jnp.dot(g16, uc, preferred_element_type=jnp.float32)

    @pl.when(c == nc - 1)
    def _epilogue():
        pre = o_ref[...]
        m = jnp.max(pre, axis=1, keepdims=True)
        lse = jnp.log(jnp.sum(jnp.exp(pre - m), axis=1, keepdims=True)) + m
        o_ref[...] = pre - lse


def _pick_bm(n):
    for bm in (400, 200, 100, 80, 40, 16, 8):
        if n % bm == 0:
            return bm
    return n


@functools.partial(jax.jit, static_argnames=())
def kernel(x, gso, W, b):
    n, f = x.shape
    bm = _pick_bm(n)
    nj = n // bm
    nc = -(-n // _CW)
    npad = nc * _CW

    w0, w1, w2 = W[0], W[1], W[2]
    wd = (w0 - w2).astype(jnp.float32)
    b2 = b.reshape(1, f).astype(jnp.float32)

    row_blk = pl.BlockSpec((bm, n), lambda j: (j, 0))
    skinny_blk = pl.BlockSpec((bm, f), lambda j: (j, 0))
    full_w = pl.BlockSpec((f, f), lambda j: (0, 0))

    u, pout = pl.pallas_call(
        functools.partial(_phase_a_body, bm=bm, n=n, f=f),
        grid=(nj,),
        in_specs=[row_blk,
                  pl.BlockSpec((n, f), lambda j: (0, 0)),
                  full_w, full_w, full_w,
                  pl.BlockSpec((1, f), lambda j: (0, 0))],
        out_specs=[skinny_blk, skinny_blk],
        out_shape=[jax.ShapeDtypeStruct((n, f), jnp.bfloat16),
                   jax.ShapeDtypeStruct((n, f), jnp.float32)],
        scratch_shapes=[pltpu.VMEM((n, 2 * f), jnp.bfloat16)],
    )(gso, x, w1, w2, wd, b2)

    u_pad = jnp.pad(u, ((0, npad - n), (0, 0)))

    out = pl.pallas_call(
        functools.partial(_phase_b_body, bm=bm, n=n, nc=nc),
        grid=(nj, nc),
        in_specs=[pl.BlockSpec((bm, _CW),
                               lambda j, c: (j, jnp.maximum(c, (j * bm) // _CW))),
                  pl.BlockSpec((npad, f), lambda j, c: (0, 0)),
                  pl.BlockSpec((bm, f), lambda j, c: (j, 0))],
        out_specs=pl.BlockSpec((bm, f), lambda j, c: (j, 0)),
        out_shape=jax.ShapeDtypeStruct((n, f), jnp.float32),
    )(gso, u_pad, pout)
    return out


# speedup vs baseline: 1.5948x; 1.2140x over previous
import functools

import numpy as np

import jax
import jax.numpy as jnp
from jax.experimental import pallas as pl
from jax.experimental.pallas import tpu as pltpu

_CW = 1280
_BMB = 1000


def _phase_a_body(g_ref, x_ref, w1_ref, w2_ref, wd_ref, b_ref,
                  u_ref, pout_ref, xu_ref, *, bm, n, f):
    j = pl.program_id(0)

    @pl.when(j == 0)
    def _init():
        xu_ref[:, :f] = x_ref[...].astype(jnp.bfloat16)
        xu_ref[:, f:] = jnp.zeros((n, f), jnp.bfloat16)

    g16 = g_ref[...].astype(jnp.bfloat16)
    both = jnp.dot(g16, xu_ref[...], preferred_element_type=jnp.float32)
    t1 = both[:, :f]
    partial = both[:, f:]

    u_j = jnp.dot(t1, w2_ref[...], preferred_element_type=jnp.float32)
    u16 = u_j.astype(jnp.bfloat16)
    xu_ref[pl.ds(j * bm, bm), f:] = u16
    u_ref[...] = u16

    x_j = xu_ref[pl.ds(j * bm, bm), :f]
    pout_ref[...] = (jnp.dot(t1, w1_ref[...], preferred_element_type=jnp.float32)
                     + jnp.dot(x_j, wd_ref[...], preferred_element_type=jnp.float32)
                     + b_ref[...]
                     + 2.0 * partial)


def _phase_b_body(jt_ref, ct_ref, g_ref, u_ref, pout_ref, o_ref,
                  *, bm, n, nc, nsteps):
    t = pl.program_id(0)
    jb = jt_ref[t]
    c = ct_ref[t]
    c_lo = (((jb * _BMB) // bm) * bm) // _CW
    bmax = (((jb + 1) * _BMB - 1) // bm) * bm

    @pl.when(c == c_lo)
    def _init():
        o_ref[...] = pout_ref[...]

    needs_mask = (c * _CW < bmax) | (c == nc - 1)

    @pl.when(needs_mask)
    def _acc_masked():
        row = jax.lax.broadcasted_iota(jnp.int32, (_BMB, _CW), 0)
        col = c * _CW + jax.lax.broadcasted_iota(jnp.int32, (_BMB, _CW), 1)
        bound = ((jb * _BMB + row) // bm) * bm
        keep = (col >= bound) & (col < n)
        g16 = jnp.where(keep, g_ref[...], 0.0).astype(jnp.bfloat16)
        uc = u_ref[pl.ds(c * _CW, _CW), :]
        o_ref[...] += 2.0 * jnp.dot(g16, uc, preferred_element_type=jnp.float32)

    @pl.when(jnp.logical_not(needs_mask))
    def _acc_raw():
        g16 = g_ref[...].astype(jnp.bfloat16)
        uc = u_ref[pl.ds(c * _CW, _CW), :]
        o_ref[...] += 2.0 * jnp.dot(g16, uc, preferred_element_type=jnp.float32)

    @pl.when(jt_ref[t + 1] != jb)
    def _epilogue():
        pre = o_ref[...]
        m = jnp.max(pre, axis=1, keepdims=True)
        lse = jnp.log(jnp.sum(jnp.exp(pre - m), axis=1, keepdims=True)) + m
        o_ref[...] = pre - lse


def _pick_bm(n):
    for bm in (400, 200, 100, 80, 40, 16, 8):
        if n % bm == 0:
            return bm
    return n


@functools.partial(jax.jit, static_argnames=())
def kernel(x, gso, W, b):
    n, f = x.shape
    bm = _pick_bm(n)
    nj = n // bm
    nc = -(-n // _CW)
    npad = nc * _CW
    njb = -(-n // _BMB)

    w0, w1, w2 = W[0], W[1], W[2]
    wd = (w0 - w2).astype(jnp.float32)
    b2 = b.reshape(1, f).astype(jnp.float32)

    row_blk = pl.BlockSpec((bm, n), lambda j: (j, 0))
    skinny_blk = pl.BlockSpec((bm, f), lambda j: (j, 0))
    full_w = pl.BlockSpec((f, f), lambda j: (0, 0))

    u, pout = pl.pallas_call(
        functools.partial(_phase_a_body, bm=bm, n=n, f=f),
        grid=(nj,),
        in_specs=[row_blk,
                  pl.BlockSpec((n, f), lambda j: (0, 0)),
                  full_w, full_w, full_w,
                  pl.BlockSpec((1, f), lambda j: (0, 0))],
        out_specs=[skinny_blk, skinny_blk],
        out_shape=[jax.ShapeDtypeStruct((n, f), jnp.bfloat16),
                   jax.ShapeDtypeStruct((n, f), jnp.float32)],
        scratch_shapes=[pltpu.VMEM((n, 2 * f), jnp.bfloat16)],
    )(gso, x, w1, w2, wd, b2)

    u_pad = jnp.pad(u, ((0, npad - n), (0, 0)))

    jt, ct = [], []
    for jb in range(njb):
        c_lo = (((jb * _BMB) // bm) * bm) // _CW
        for c in range(c_lo, nc):
            jt.append(jb)
            ct.append(c)
    nsteps = len(jt)
    jt.append(-1)
    ct.append(0)
    jt_arr = jnp.asarray(np.asarray(jt, np.int32))
    ct_arr = jnp.asarray(np.asarray(ct, np.int32))

    grid_spec = pltpu.PrefetchScalarGridSpec(
        num_scalar_prefetch=2,
        grid=(nsteps,),
        in_specs=[
            pl.BlockSpec((_BMB, _CW), lambda t, jt, ct: (jt[t], ct[t])),
            pl.BlockSpec((npad, f), lambda t, jt, ct: (0, 0)),
            pl.BlockSpec((_BMB, f), lambda t, jt, ct: (jt[t], 0)),
        ],
        out_specs=pl.BlockSpec((_BMB, f), lambda t, jt, ct: (jt[t], 0)),
    )

    out = pl.pallas_call(
        functools.partial(_phase_b_body, bm=bm, n=n, nc=nc, nsteps=nsteps),
        grid_spec=grid_spec,
        out_shape=jax.ShapeDtypeStruct((n, f), jnp.float32),
    )(jt_arr, ct_arr, gso, u_pad, pout)
    return out


# speedup vs baseline: 1.6350x; 1.0252x over previous
import functools

import numpy as np

import jax
import jax.numpy as jnp
from jax.experimental import pallas as pl
from jax.experimental.pallas import tpu as pltpu

_CW = 2560
_BMB = 1000


def _phase_a_body(g_ref, x_ref, w1_ref, w2_ref, wd_ref, b_ref,
                  u_ref, pout_ref, xu_ref, *, bm, n, f):
    j = pl.program_id(0)

    @pl.when(j == 0)
    def _init():
        xu_ref[:, :f] = x_ref[...].astype(jnp.bfloat16)
        xu_ref[:, f:] = jnp.zeros((n, f), jnp.bfloat16)

    g16 = g_ref[...].astype(jnp.bfloat16)
    both = jnp.dot(g16, xu_ref[...], preferred_element_type=jnp.float32)
    t1 = both[:, :f]
    partial = both[:, f:]

    u_j = jnp.dot(t1, w2_ref[...], preferred_element_type=jnp.float32)
    u16 = u_j.astype(jnp.bfloat16)
    xu_ref[pl.ds(j * bm, bm), f:] = u16
    u_ref[...] = u16

    x_j = xu_ref[pl.ds(j * bm, bm), :f]
    pout_ref[...] = (jnp.dot(t1, w1_ref[...], preferred_element_type=jnp.float32)
                     + jnp.dot(x_j, wd_ref[...], preferred_element_type=jnp.float32)
                     + b_ref[...]
                     + 2.0 * partial)


def _phase_b_body(jt_ref, ct_ref, g_ref, u_ref, pout_ref, o_ref,
                  *, bm, n, nc, nsteps):
    t = pl.program_id(0)
    jb = jt_ref[t]
    c = ct_ref[t]
    c_lo = (((jb * _BMB) // bm) * bm) // _CW
    bmax = (((jb + 1) * _BMB - 1) // bm) * bm

    @pl.when(c == c_lo)
    def _init():
        o_ref[...] = pout_ref[...]

    needs_mask = (c * _CW < bmax) | (c == nc - 1)

    @pl.when(needs_mask)
    def _acc_masked():
        row = jax.lax.broadcasted_iota(jnp.int32, (_BMB, _CW), 0)
        col = c * _CW + jax.lax.broadcasted_iota(jnp.int32, (_BMB, _CW), 1)
        bound = ((jb * _BMB + row) // bm) * bm
        keep = (col >= bound) & (col < n)
        g16 = jnp.where(keep, g_ref[...], 0.0).astype(jnp.bfloat16)
        uc = u_ref[pl.ds(c * _CW, _CW), :]
        o_ref[...] += 2.0 * jnp.dot(g16, uc, preferred_element_type=jnp.float32)

    @pl.when(jnp.logical_not(needs_mask))
    def _acc_raw():
        g16 = g_ref[...].astype(jnp.bfloat16)
        uc = u_ref[pl.ds(c * _CW, _CW), :]
        o_ref[...] += 2.0 * jnp.dot(g16, uc, preferred_element_type=jnp.float32)

    @pl.when(jt_ref[t + 1] != jb)
    def _epilogue():
        pre = o_ref[...]
        m = jnp.max(pre, axis=1, keepdims=True)
        lse = jnp.log(jnp.sum(jnp.exp(pre - m), axis=1, keepdims=True)) + m
        o_ref[...] = pre - lse


def _pick_bm(n):
    for bm in (400, 200, 100, 80, 40, 16, 8):
        if n % bm == 0:
            return bm
    return n


@functools.partial(jax.jit, static_argnames=())
def kernel(x, gso, W, b):
    n, f = x.shape
    bm = _pick_bm(n)
    nj = n // bm
    nc = -(-n // _CW)
    npad = nc * _CW
    njb = -(-n // _BMB)

    w0, w1, w2 = W[0], W[1], W[2]
    wd = (w0 - w2).astype(jnp.float32)
    b2 = b.reshape(1, f).astype(jnp.float32)

    row_blk = pl.BlockSpec((bm, n), lambda j: (j, 0))
    skinny_blk = pl.BlockSpec((bm, f), lambda j: (j, 0))
    full_w = pl.BlockSpec((f, f), lambda j: (0, 0))

    u, pout = pl.pallas_call(
        functools.partial(_phase_a_body, bm=bm, n=n, f=f),
        grid=(nj,),
        in_specs=[row_blk,
                  pl.BlockSpec((n, f), lambda j: (0, 0)),
                  full_w, full_w, full_w,
                  pl.BlockSpec((1, f), lambda j: (0, 0))],
        out_specs=[skinny_blk, skinny_blk],
        out_shape=[jax.ShapeDtypeStruct((n, f), jnp.bfloat16),
                   jax.ShapeDtypeStruct((n, f), jnp.float32)],
        scratch_shapes=[pltpu.VMEM((n, 2 * f), jnp.bfloat16)],
    )(gso, x, w1, w2, wd, b2)

    u_pad = jnp.pad(u, ((0, npad - n), (0, 0)))

    jt, ct = [], []
    for jb in range(njb):
        c_lo = (((jb * _BMB) // bm) * bm) // _CW
        for c in range(c_lo, nc):
            jt.append(jb)
            ct.append(c)
    nsteps = len(jt)
    jt.append(-1)
    ct.append(0)
    jt_arr = jnp.asarray(np.asarray(jt, np.int32))
    ct_arr = jnp.asarray(np.asarray(ct, np.int32))

    grid_spec = pltpu.PrefetchScalarGridSpec(
        num_scalar_prefetch=2,
        grid=(nsteps,),
        in_specs=[
            pl.BlockSpec((_BMB, _CW), lambda t, jt, ct: (jt[t], ct[t])),
            pl.BlockSpec((npad, f), lambda t, jt, ct: (0, 0)),
            pl.BlockSpec((_BMB, f), lambda t, jt, ct: (jt[t], 0)),
        ],
        out_specs=pl.BlockSpec((_BMB, f), lambda t, jt, ct: (jt[t], 0)),
    )

    out = pl.pallas_call(
        functools.partial(_phase_b_body, bm=bm, n=n, nc=nc, nsteps=nsteps),
        grid_spec=grid_spec,
        out_shape=jax.ShapeDtypeStruct((n, f), jnp.float32),
    )(jt_arr, ct_arr, gso, u_pad, pout)
    return out
